# baseline (speedup 1.0000x reference)
import functools

import jax
import jax.numpy as jnp
from jax import lax
from jax.experimental import pallas as pl
from jax.experimental.pallas import tpu as pltpu
from jax.experimental.pallas import tpu_sc as plsc


def _sc_gather(tab_a, tab_b, t):
    B = t.shape[0]
    try:
        info = plsc.get_sparse_core_info()
        NC, NS, L = info.num_cores, info.num_subcores, info.num_lanes
    except Exception:
        NC, NS, L = 2, 16, 16
    NW = NC * NS
    b_per_w = B // NW

    mesh = plsc.VectorSubcoreMesh(core_axis_name="c", subcore_axis_name="s")

    @functools.partial(
        pl.kernel,
        out_type=(
            jax.ShapeDtypeStruct((B,), jnp.float32),
            jax.ShapeDtypeStruct((B,), jnp.float32),
        ),
        mesh=mesh,
        scratch_types=[
            pltpu.VMEM((b_per_w,), jnp.int32),
            pltpu.VMEM((b_per_w,), jnp.float32),
            pltpu.VMEM((b_per_w,), jnp.float32),
            pltpu.SemaphoreType.DMA,
            pltpu.SemaphoreType.DMA,
        ],
    )
    def k(tab_a_hbm, tab_b_hbm, t_hbm, out_a_hbm, out_b_hbm,
          idx_v, va_v, vb_v, sem_a, sem_b):
        wid = lax.axis_index("s") * NC + lax.axis_index("c")
        base = wid * b_per_w
        pltpu.sync_copy(t_hbm.at[pl.ds(base, b_per_w)], idx_v)
        cp_a = pltpu.async_copy(tab_a_hbm.at[idx_v], va_v, sem_a)
        cp_b = pltpu.async_copy(tab_b_hbm.at[idx_v], vb_v, sem_b)
        cp_a.wait()
        cp_b.wait()
        pltpu.sync_copy(va_v, out_a_hbm.at[pl.ds(base, b_per_w)])
        pltpu.sync_copy(vb_v, out_b_hbm.at[pl.ds(base, b_per_w)])

    return k(tab_a, tab_b, t)


def _blend_body(a_ref, b_ref, x_ref, n_ref, o_ref):
    o_ref[...] = a_ref[...] * x_ref[...] + b_ref[...] * n_ref[...]


def _blend(a2, b2, x2, n2, row_block):
    B, L = x2.shape
    return pl.pallas_call(
        _blend_body,
        grid=(B // row_block,),
        in_specs=[
            pl.BlockSpec((row_block, 1), lambda i: (i, 0)),
            pl.BlockSpec((row_block, 1), lambda i: (i, 0)),
            pl.BlockSpec((row_block, L), lambda i: (i, 0)),
            pl.BlockSpec((row_block, L), lambda i: (i, 0)),
        ],
        out_specs=pl.BlockSpec((row_block, L), lambda i: (i, 0)),
        out_shape=jax.ShapeDtypeStruct((B, L), jnp.float32),
    )(a2, b2, x2, n2)


def kernel(x_0, t, noise, sqrt_alphas_cumprod, sqrt_one_minus_alphas_cumprod):
    B, L = x_0.shape[0], x_0.shape[1]
    a, b = _sc_gather(sqrt_alphas_cumprod, sqrt_one_minus_alphas_cumprod,
                      t.astype(jnp.int32))
    x2 = x_0.reshape(B, L)
    n2 = noise.reshape(B, L)
    xt = _blend(a.reshape(B, 1), b.reshape(B, 1), x2, n2, row_block=256)
    return xt.reshape(B, L, 1), noise


# speedup vs baseline: 2.7539x; 2.7539x over previous
import functools

import jax
import jax.numpy as jnp
from jax import lax
from jax.experimental import pallas as pl
from jax.experimental.pallas import tpu as pltpu
from jax.experimental.pallas import tpu_sc as plsc


def _sc_gather(tab_a, tab_b, t):
    B = t.shape[0]
    try:
        info = plsc.get_sparse_core_info()
        NC, NS, L = info.num_cores, info.num_subcores, info.num_lanes
    except Exception:
        NC, NS, L = 2, 16, 16
    NW = NC * NS
    b_per_w = B // NW

    mesh = plsc.VectorSubcoreMesh(core_axis_name="c", subcore_axis_name="s")

    @functools.partial(
        pl.kernel,
        out_type=(
            jax.ShapeDtypeStruct((B,), jnp.float32),
            jax.ShapeDtypeStruct((B,), jnp.float32),
        ),
        mesh=mesh,
        scratch_types=[
            pltpu.VMEM((b_per_w,), jnp.int32),
            pltpu.VMEM((b_per_w,), jnp.float32),
            pltpu.VMEM((b_per_w,), jnp.float32),
            pltpu.SemaphoreType.DMA,
            pltpu.SemaphoreType.DMA,
        ],
    )
    def k(tab_a_hbm, tab_b_hbm, t_hbm, out_a_hbm, out_b_hbm,
          idx_v, va_v, vb_v, sem_a, sem_b):
        wid = lax.axis_index("s") * NC + lax.axis_index("c")
        base = wid * b_per_w
        pltpu.sync_copy(t_hbm.at[pl.ds(base, b_per_w)], idx_v)
        cp_a = pltpu.async_copy(tab_a_hbm.at[idx_v], va_v, sem_a)
        cp_b = pltpu.async_copy(tab_b_hbm.at[idx_v], vb_v, sem_b)
        cp_a.wait()
        cp_b.wait()
        pltpu.sync_copy(va_v, out_a_hbm.at[pl.ds(base, b_per_w)])
        pltpu.sync_copy(vb_v, out_b_hbm.at[pl.ds(base, b_per_w)])

    return k(tab_a, tab_b, t)


def _blend_body(a_ref, b_ref, x_ref, n_ref, o_ref, no_ref):
    nvals = n_ref[...]
    o_ref[...] = a_ref[...] * x_ref[...] + b_ref[...] * nvals
    no_ref[...] = nvals


def _blend(a3, b3, x3, n3, row_block):
    B, S, LN = x3.shape
    scale_spec = pl.BlockSpec((row_block, 1, 1), lambda i: (i, 0, 0))
    data_spec = pl.BlockSpec((row_block, S, LN), lambda i: (i, 0, 0))
    return pl.pallas_call(
        _blend_body,
        grid=(B // row_block,),
        in_specs=[scale_spec, scale_spec, data_spec, data_spec],
        out_specs=[data_spec, data_spec],
        out_shape=[
            jax.ShapeDtypeStruct((B, S, LN), jnp.float32),
            jax.ShapeDtypeStruct((B, S, LN), jnp.float32),
        ],
    )(a3, b3, x3, n3)


def kernel(x_0, t, noise, sqrt_alphas_cumprod, sqrt_one_minus_alphas_cumprod):
    B, L = x_0.shape[0], x_0.shape[1]
    a, b = _sc_gather(sqrt_alphas_cumprod, sqrt_one_minus_alphas_cumprod,
                      t.astype(jnp.int32))
    x3 = x_0.reshape(B, L // 128, 128)
    n3 = noise.reshape(B, L // 128, 128)
    xt3, no3 = _blend(a.reshape(B, 1, 1), b.reshape(B, 1, 1), x3, n3,
                      row_block=128)
    return xt3.reshape(B, L, 1), no3.reshape(B, L, 1)


# speedup vs baseline: 2.7831x; 1.0106x over previous
import functools

import jax
import jax.numpy as jnp
from jax import lax
from jax.experimental import pallas as pl
from jax.experimental.pallas import tpu as pltpu
from jax.experimental.pallas import tpu_sc as plsc


def _sc_gather(tab_a, tab_b, t):
    B = t.shape[0]
    try:
        info = plsc.get_sparse_core_info()
        NC, NS, L = info.num_cores, info.num_subcores, info.num_lanes
    except Exception:
        NC, NS, L = 2, 16, 16
    NW = NC * NS
    b_per_w = B // NW

    mesh = plsc.VectorSubcoreMesh(core_axis_name="c", subcore_axis_name="s")

    @functools.partial(
        pl.kernel,
        out_type=(
            jax.ShapeDtypeStruct((B,), jnp.float32),
            jax.ShapeDtypeStruct((B,), jnp.float32),
        ),
        mesh=mesh,
        scratch_types=[
            pltpu.VMEM((b_per_w,), jnp.int32),
            pltpu.VMEM((b_per_w,), jnp.float32),
            pltpu.VMEM((b_per_w,), jnp.float32),
            pltpu.SemaphoreType.DMA,
            pltpu.SemaphoreType.DMA,
        ],
    )
    def k(tab_a_hbm, tab_b_hbm, t_hbm, out_a_hbm, out_b_hbm,
          idx_v, va_v, vb_v, sem_a, sem_b):
        wid = lax.axis_index("s") * NC + lax.axis_index("c")
        base = wid * b_per_w
        pltpu.sync_copy(t_hbm.at[pl.ds(base, b_per_w)], idx_v)
        cp_a = pltpu.async_copy(tab_a_hbm.at[idx_v], va_v, sem_a)
        cp_b = pltpu.async_copy(tab_b_hbm.at[idx_v], vb_v, sem_b)
        cp_a.wait()
        cp_b.wait()
        pltpu.sync_copy(va_v, out_a_hbm.at[pl.ds(base, b_per_w)])
        pltpu.sync_copy(vb_v, out_b_hbm.at[pl.ds(base, b_per_w)])

    return k(tab_a, tab_b, t)


def _blend_body(a_ref, b_ref, x_ref, n_ref, o_ref, no_ref):
    nvals = n_ref[...]
    o_ref[...] = a_ref[...] * x_ref[...] + b_ref[...] * nvals
    no_ref[...] = nvals


def _blend(a3, b3, x3, n3, row_block):
    B, S, LN = x3.shape
    scale_spec = pl.BlockSpec((row_block, 1, 1), lambda i: (i, 0, 0))
    data_spec = pl.BlockSpec((row_block, S, LN), lambda i: (i, 0, 0))
    return pl.pallas_call(
        _blend_body,
        grid=(B // row_block,),
        in_specs=[scale_spec, scale_spec, data_spec, data_spec],
        out_specs=[data_spec, data_spec],
        out_shape=[
            jax.ShapeDtypeStruct((B, S, LN), jnp.float32),
            jax.ShapeDtypeStruct((B, S, LN), jnp.float32),
        ],
    )(a3, b3, x3, n3)


def kernel(x_0, t, noise, sqrt_alphas_cumprod, sqrt_one_minus_alphas_cumprod):
    B, L = x_0.shape[0], x_0.shape[1]
    a, b = _sc_gather(sqrt_alphas_cumprod, sqrt_one_minus_alphas_cumprod,
                      t.astype(jnp.int32))
    x3 = x_0.reshape(B, L // 128, 128)
    n3 = noise.reshape(B, L // 128, 128)
    xt3, no3 = _blend(a.reshape(B, 1, 1), b.reshape(B, 1, 1), x3, n3,
                      row_block=256)
    return xt3.reshape(B, L, 1), no3.reshape(B, L, 1)
